# baseline (speedup 1.0000x reference)
import functools

import jax
import jax.numpy as jnp
from jax import lax
from jax.experimental import pallas as pl
from jax.experimental.pallas import tpu as pltpu
from jax.experimental.pallas import tpu_sc as plsc

_info = plsc.get_sparse_core_info()
_NC, _NS, _L = _info.num_cores, _info.num_subcores, _info.num_lanes
_NW = _NC * _NS


def _make_sc_score(batch, dim):
  half = dim // 2
  bpw = batch // _NW
  groups = bpw // _L
  mesh = plsc.VectorSubcoreMesh(core_axis_name="c", subcore_axis_name="s")

  @functools.partial(
      pl.kernel,
      mesh=mesh,
      out_type=jax.ShapeDtypeStruct((batch,), jnp.float32),
      compiler_params=pltpu.CompilerParams(needs_layout_passes=False),
      scratch_types=[
          pltpu.VMEM((bpw, 3), jnp.int32),
          pltpu.VMEM((bpw,), jnp.int32),
          pltpu.VMEM((bpw,), jnp.int32),
          pltpu.VMEM((bpw,), jnp.int32),
          pltpu.VMEM((bpw, dim), jnp.float32),
          pltpu.VMEM((bpw, dim), jnp.float32),
          pltpu.VMEM((bpw, dim), jnp.float32),
          pltpu.VMEM((bpw,), jnp.float32),
          pltpu.SemaphoreType.DMA,
      ],
  )
  def sc_score(samp_hbm, ent_hbm, rel_hbm, out_hbm,
               samp_v, hidx_v, ridx_v, tidx_v, hrow_v, rrow_v, trow_v,
               out_v, sem):
    wid = lax.axis_index("s") * _NC + lax.axis_index("c")
    base = wid * bpw
    pltpu.sync_copy(samp_hbm.at[pl.ds(base, bpw)], samp_v)

    lane0 = lax.iota(jnp.int32, _L)
    rows0 = [g * _L + lane0 for g in range(bpw // _L)]
    for g in range(bpw // _L):
      for col, buf in ((0, hidx_v), (1, ridx_v), (2, tidx_v)):
        col_v = jnp.full((_L,), col, jnp.int32)
        buf[pl.ds(g * _L, _L)] = plsc.load_gather(samp_v, [rows0[g], col_v])

    ch = pltpu.async_copy(ent_hbm.at[hidx_v], hrow_v, sem)
    cr = pltpu.async_copy(rel_hbm.at[ridx_v], rrow_v, sem)
    ct = pltpu.async_copy(ent_hbm.at[tidx_v], trow_v, sem)
    ch.wait()
    cr.wait()
    ct.wait()

    lane = lane0
    rows = rows0

    def dim_body(d, accs):
      rot = (lane + d) & (half - 1)
      im_col = rot + half
      out = []
      for g in range(groups):
        rh = plsc.load_gather(hrow_v, [rows[g], rot])
        ih = plsc.load_gather(hrow_v, [rows[g], im_col])
        rr = plsc.load_gather(rrow_v, [rows[g], rot])
        ir = plsc.load_gather(rrow_v, [rows[g], im_col])
        rt = plsc.load_gather(trow_v, [rows[g], rot])
        it = plsc.load_gather(trow_v, [rows[g], im_col])
        out.append(accs[g] + (rh * rr - ih * ir) * rt
                   + (rh * ir + ih * rr) * it)
      return tuple(out)

    accs = lax.fori_loop(
        0, half, dim_body,
        tuple(jnp.zeros((_L,), jnp.float32) for _ in range(groups)))
    for g in range(groups):
      out_v[pl.ds(g * _L, _L)] = accs[g]

    pltpu.sync_copy(out_v, out_hbm.at[pl.ds(base, bpw)])

  return sc_score


def kernel(sample, entity_embedding, relation_embedding):
  batch = sample.shape[0]
  dim = entity_embedding.shape[1]
  score = _make_sc_score(batch, dim)(
      sample, entity_embedding, relation_embedding)
  return score.reshape(batch, 1)


# speedup vs baseline: 1.0827x; 1.0827x over previous
import functools

import jax
import jax.numpy as jnp
from jax import lax
from jax.experimental import pallas as pl
from jax.experimental.pallas import tpu as pltpu
from jax.experimental.pallas import tpu_sc as plsc

_info = plsc.get_sparse_core_info()
_NC, _NS, _L = _info.num_cores, _info.num_subcores, _info.num_lanes
_NW = _NC * _NS


def _make_sc_score(batch, dim):
  half = dim // 2
  bpw = batch // _NW
  hb = bpw // 2
  gph = hb // _L
  mesh = plsc.VectorSubcoreMesh(core_axis_name="c", subcore_axis_name="s")

  @functools.partial(
      pl.kernel,
      mesh=mesh,
      out_type=jax.ShapeDtypeStruct((batch,), jnp.float32),
      compiler_params=pltpu.CompilerParams(needs_layout_passes=False),
      scratch_types=[
          pltpu.VMEM((bpw,), jnp.int32),
          pltpu.VMEM((bpw,), jnp.int32),
          pltpu.VMEM((bpw,), jnp.int32),
          pltpu.VMEM((bpw, dim), jnp.float32),
          pltpu.VMEM((bpw, dim), jnp.float32),
          pltpu.VMEM((bpw, dim), jnp.float32),
          pltpu.VMEM((bpw,), jnp.float32),
          pltpu.SemaphoreType.DMA,
          pltpu.SemaphoreType.DMA,
          pltpu.SemaphoreType.DMA,
      ],
  )
  def sc_score(hidx_hbm, ridx_hbm, tidx_hbm, ent_hbm, rel_hbm, out_hbm,
               hidx_v, ridx_v, tidx_v, hrow_v, rrow_v, trow_v, out_v,
               semi, sem0, sem1):
    wid = lax.axis_index("s") * _NC + lax.axis_index("c")
    base = wid * bpw
    ci = (pltpu.async_copy(hidx_hbm.at[pl.ds(base, bpw)], hidx_v, semi),
          pltpu.async_copy(ridx_hbm.at[pl.ds(base, bpw)], ridx_v, semi),
          pltpu.async_copy(tidx_hbm.at[pl.ds(base, bpw)], tidx_v, semi))
    for c in ci:
      c.wait()

    sems = (sem0, sem1)
    copies = []
    for h in range(2):
      sl = pl.ds(h * hb, hb)
      copies.append((
          pltpu.async_copy(ent_hbm.at[hidx_v.at[sl]], hrow_v.at[sl], sems[h]),
          pltpu.async_copy(rel_hbm.at[ridx_v.at[sl]], rrow_v.at[sl], sems[h]),
          pltpu.async_copy(ent_hbm.at[tidx_v.at[sl]], trow_v.at[sl], sems[h]),
      ))

    lane = lax.iota(jnp.int32, _L)

    for h in range(2):
      for c in copies[h]:
        c.wait()
      rows = [h * hb + g * _L + lane for g in range(gph)]

      def dim_body(d, accs, rows=rows):
        rot = (lane + d) & (half - 1)
        im_col = rot + half
        out = []
        for g in range(gph):
          rh = plsc.load_gather(hrow_v, [rows[g], rot])
          ih = plsc.load_gather(hrow_v, [rows[g], im_col])
          rr = plsc.load_gather(rrow_v, [rows[g], rot])
          ir = plsc.load_gather(rrow_v, [rows[g], im_col])
          rt = plsc.load_gather(trow_v, [rows[g], rot])
          it = plsc.load_gather(trow_v, [rows[g], im_col])
          out.append(accs[g] + (rh * rr - ih * ir) * rt
                     + (rh * ir + ih * rr) * it)
        return tuple(out)

      accs = lax.fori_loop(
          0, half, dim_body,
          tuple(jnp.zeros((_L,), jnp.float32) for _ in range(gph)))
      for g in range(gph):
        out_v[pl.ds(h * hb + g * _L, _L)] = accs[g]

    pltpu.sync_copy(out_v, out_hbm.at[pl.ds(base, bpw)])

  return sc_score


def kernel(sample, entity_embedding, relation_embedding):
  batch = sample.shape[0]
  dim = entity_embedding.shape[1]
  hidx = sample[:, 0]
  ridx = sample[:, 1]
  tidx = sample[:, 2]
  score = _make_sc_score(batch, dim)(
      hidx, ridx, tidx, entity_embedding, relation_embedding)
  return score.reshape(batch, 1)
